# baseline (speedup 1.0000x reference)
import functools

import jax
import jax.numpy as jnp
from jax import lax
from jax.experimental import pallas as pl
from jax.experimental.pallas import tpu as pltpu
from jax.experimental.pallas import tpu_sc as plsc

B, C, H, W = 32, 2, 152, 272
HW = H * W
K = 500
KP = 512
VT = 608
NS, L = 16, 16
NCHUNK = KP // L
NB = 1


def _sc_body(flow_hbm, tab_hbm, out_hbm, *refs):
    bufs = [refs[12 * r:12 * r + 12] for r in range(NB)]
    acc_v, red_v, out_v, shared, sem_in, sem_g = refs[12 * NB:]
    core = lax.axis_index("c")
    s = lax.axis_index("s")
    lane = lax.iota(jnp.int32, L)
    ones = jnp.ones((L,), jnp.int32)

    if True:
        ind = []
        for r in range(NB):
            b = s * 2 + core
            for a in range(5):
                ind.append(pltpu.async_copy(tab_hbm.at[a, b], bufs[r][a], sem_in))

        def init_body(t, _):
            for r in range(NB):
                bufs[r][5][pl.ds(t * L, L)] = jnp.zeros((L,), jnp.int32)
                bufs[r][6][pl.ds(t * L, L)] = jnp.full((L,), -1, jnp.int32)
            return 0
        lax.fori_loop(0, VT // L, init_body, 0)

        with jax.named_scope("ph_stage_wait"):
            for d in ind:
                d.wait()

        def idx_body(t, macc):
            sl = pl.ds(t * L, L)
            out = []
            for r in range(NB):
                b = s * 2 + core
                index_v, mask_v = bufs[r][2], bufs[r][4]
                g0_v, g1_v = bufs[r][7], bufs[r][8]
                base0 = b * (C * HW)
                hw = index_v[sl]
                g0_v[sl] = hw + base0
                g1_v[sl] = hw + (base0 + HW)
                out.append(macc[r] + mask_v[sl])
            return tuple(out)
        with jax.named_scope("ph_idx"):
            msums = lax.fori_loop(0, NCHUNK, idx_body,
                                  (jnp.zeros((L,), jnp.int32),) * NB)

        gd = []
        for r in range(NB):
            g0_v, g1_v, f0_v, f1_v = bufs[r][7:11]
            for i in range(KP // 128):
                gd.append(pltpu.async_copy(
                    flow_hbm.at[g0_v.at[pl.ds(i * 128, 128)]],
                    f0_v.at[pl.ds(i * 128, 128)], sem_g))
                gd.append(pltpu.async_copy(
                    flow_hbm.at[g1_v.at[pl.ds(i * 128, 128)]],
                    f1_v.at[pl.ds(i * 128, 128)], sem_g))

        def table_body(t, _):
            sl = pl.ds(t * L, L)
            for r in range(NB):
                ids_v, ids2_v = bufs[r][0], bufs[r][1]
                cnt1, last2 = bufs[r][5], bufs[r][6]
                plsc.addupdate_scatter(cnt1, [ids_v[sl]], ones)
                plsc.store_scatter(last2, [ids2_v[sl]], t * L + lane)
            return 0
        with jax.named_scope("ph_tables"):
            lax.fori_loop(0, NCHUNK, table_body, 0)

        with jax.named_scope("ph_gdrain"):
            for d in gd:
                d.wait()

        def eval_body(t, nacc):
            sl = pl.ds(t * L, L)
            for r in range(NB):
                (ids_v, ids2_v, index_v, index2_v, mask_v,
                 cnt1, last2, g0_v, g1_v, f0_v, f1_v, srt_v) = bufs[r]
                v = ids_v[sl]
                c1 = plsc.load_gather(cnt1, [v])
                j2 = plsc.load_gather(last2, [v])
                wr = (v != 0) & (c1 == 1) & (j2 >= 0)
                idx2 = plsc.load_gather(index2_v, [jnp.maximum(j2, 0)])
                r1i = ((idx2.astype(jnp.float32) + 0.5)
                       * jnp.float32(1.0 / W)).astype(jnp.int32)
                r1 = r1i.astype(jnp.float32)
                r0 = (idx2 - r1i * W).astype(jnp.float32)
                m = mask_v[sl].astype(jnp.float32)
                term = (jnp.abs(f0_v[sl] * m - r0)
                        + jnp.abs(f1_v[sl] * m - r1))
                nacc = nacc + jnp.where(wr, term, 0.0)
            return nacc
        with jax.named_scope("ph_eval"):
            numer = lax.fori_loop(0, NCHUNK, eval_body, jnp.zeros((L,), jnp.float32))

        jax.named_scope("ph_combine").__enter__()
        ns = jnp.sum(numer)
        ms = jnp.sum(msums[0]).astype(jnp.float32)
        acc_v[:] = jnp.where(lane == 0, ns, jnp.where(lane == 1, ms, 0.0))
        pltpu.sync_copy(acc_v, shared.at[pl.ds(s * L, L)])
        plsc.subcore_barrier()

        @pl.when(s == 0)
        def _():
            pltpu.sync_copy(shared, red_v)

            def red_body(i, tot):
                return tot + red_v[pl.ds(i * L, L)]
            tot = lax.fori_loop(0, NS, red_body, jnp.zeros((L,), jnp.float32))
            acc_v[:] = tot
            pltpu.sync_copy(acc_v, out_hbm.at[core])


@jax.jit
def kernel(flow, mask, index, ids, index2, ids2):
    tab = jnp.pad(jnp.stack([ids, ids2, index, index2, mask]),
                  ((0, 0), (0, 0), (0, KP - K)))
    flow_flat = flow.reshape(-1)

    per_batch = [
        pltpu.VMEM((KP,), jnp.int32),
        pltpu.VMEM((KP,), jnp.int32),
        pltpu.VMEM((KP,), jnp.int32),
        pltpu.VMEM((KP,), jnp.int32),
        pltpu.VMEM((KP,), jnp.int32),
        pltpu.VMEM((VT,), jnp.int32),
        pltpu.VMEM((VT,), jnp.int32),
        pltpu.VMEM((KP,), jnp.int32),
        pltpu.VMEM((KP,), jnp.int32),
        pltpu.VMEM((KP,), jnp.float32),
        pltpu.VMEM((KP,), jnp.float32),
        pltpu.VMEM((L,), jnp.int32),
    ]
    mesh = plsc.VectorSubcoreMesh(core_axis_name="c", subcore_axis_name="s")
    run = functools.partial(
        pl.kernel, mesh=mesh,
        compiler_params=pltpu.CompilerParams(needs_layout_passes=False),
        out_type=jax.ShapeDtypeStruct((2, L), jnp.float32),
        scratch_types=per_batch * NB + [
            pltpu.VMEM((L,), jnp.float32),
            pltpu.VMEM((NS * L,), jnp.float32),
            pltpu.VMEM((L,), jnp.float32),
            pltpu.VMEM_SHARED((NS * L,), jnp.float32),
            pltpu.SemaphoreType.DMA,
            pltpu.SemaphoreType.DMA,
        ],
    )(_sc_body)
    out = run(flow_flat, tab)
    return (out[0, 0] + out[1, 0]) / (2.0 * (out[0, 1] + out[1, 1]) + 0.0001)


# speedup vs baseline: 1.0769x; 1.0769x over previous
import functools

import jax
import jax.numpy as jnp
from jax import lax
from jax.experimental import pallas as pl
from jax.experimental.pallas import tpu as pltpu
from jax.experimental.pallas import tpu_sc as plsc

B, C, H, W = 32, 2, 152, 272
HW = H * W
K = 500
KP = 512
VT = 608
NS, L = 16, 16
NCHUNK = KP // L
NB = 2


def _sc_body(flow_hbm, tab_hbm, out_hbm, *refs):
    bufs = [refs[12 * r:12 * r + 12] for r in range(NB)]
    acc_v, red_v, out_v, shared, sem_in, sem_g = refs[12 * NB:]
    core = lax.axis_index("c")
    s = lax.axis_index("s")
    lane = lax.iota(jnp.int32, L)
    ones = jnp.ones((L,), jnp.int32)

    @pl.when(core == 0)
    def _():
        ind = {}
        for r in range(NB):
            b = s * NB + r
            for a in (2, 0, 1, 3, 4):
                ind[(r, a)] = pltpu.async_copy(tab_hbm.at[a, b], bufs[r][a],
                                               sem_in)
        for r in range(NB):
            ind[(r, 2)].wait()

        def idx_body(t, _):
            sl = pl.ds(t * L, L)
            for r in range(NB):
                b = s * NB + r
                index_v = bufs[r][2]
                g0_v, g1_v = bufs[r][7], bufs[r][8]
                base0 = b * (C * HW)
                hw = index_v[sl]
                g0_v[sl] = hw + base0
                g1_v[sl] = hw + (base0 + HW)
            return 0
        lax.fori_loop(0, NCHUNK, idx_body, 0)

        gd = []
        for r in range(NB):
            g0_v, g1_v, f0_v, f1_v = bufs[r][7:11]
            for i in range(KP // 128):
                gd.append(pltpu.async_copy(
                    flow_hbm.at[g0_v.at[pl.ds(i * 128, 128)]],
                    f0_v.at[pl.ds(i * 128, 128)], sem_g))
                gd.append(pltpu.async_copy(
                    flow_hbm.at[g1_v.at[pl.ds(i * 128, 128)]],
                    f1_v.at[pl.ds(i * 128, 128)], sem_g))

        def init_body(t, _):
            for r in range(NB):
                bufs[r][5][pl.ds(t * L, L)] = jnp.zeros((L,), jnp.int32)
                bufs[r][6][pl.ds(t * L, L)] = jnp.full((L,), -1, jnp.int32)
            return 0
        lax.fori_loop(0, VT // L, init_body, 0)
        for r in range(NB):
            for a in (0, 1, 3, 4):
                ind[(r, a)].wait()

        def table_body(t, _):
            sl = pl.ds(t * L, L)
            for r in range(NB):
                ids_v, ids2_v = bufs[r][0], bufs[r][1]
                cnt1, last2 = bufs[r][5], bufs[r][6]
                plsc.addupdate_scatter(cnt1, [ids_v[sl]], ones)
                plsc.store_scatter(last2, [ids2_v[sl]], t * L + lane)
            return 0
        lax.fori_loop(0, NCHUNK, table_body, 0)

        for d in gd:
            d.wait()

        def eval_body(t, carry):
            nacc, macc = carry
            sl = pl.ds(t * L, L)
            for r in range(NB):
                (ids_v, ids2_v, index_v, index2_v, mask_v,
                 cnt1, last2, g0_v, g1_v, f0_v, f1_v, srt_v) = bufs[r]
                v = ids_v[sl]
                c1 = plsc.load_gather(cnt1, [v])
                j2 = plsc.load_gather(last2, [v])
                wr = (v != 0) & (c1 == 1) & (j2 >= 0)
                idx2 = plsc.load_gather(index2_v, [jnp.maximum(j2, 0)])
                r1i = ((idx2.astype(jnp.float32) + 0.5)
                       * jnp.float32(1.0 / W)).astype(jnp.int32)
                r1 = r1i.astype(jnp.float32)
                r0 = (idx2 - r1i * W).astype(jnp.float32)
                m = mask_v[sl].astype(jnp.float32)
                term = (jnp.abs(f0_v[sl] * m - r0)
                        + jnp.abs(f1_v[sl] * m - r1))
                nacc = nacc + jnp.where(wr, term, 0.0)
                macc = macc + mask_v[sl]
            return nacc, macc
        numer, msum = lax.fori_loop(
            0, NCHUNK, eval_body,
            (jnp.zeros((L,), jnp.float32), jnp.zeros((L,), jnp.int32)))

        ns = jnp.sum(numer)
        ms = jnp.sum(msum).astype(jnp.float32)
        acc_v[:] = jnp.where(lane == 0, ns, jnp.where(lane == 1, ms, 0.0))
        pltpu.sync_copy(acc_v, shared.at[pl.ds(s * L, L)])
        plsc.subcore_barrier()

        @pl.when(s == 0)
        def _():
            pltpu.sync_copy(shared, red_v)

            def red_body(i, tot):
                return tot + red_v[pl.ds(i * L, L)]
            tot = lax.fori_loop(0, NS, red_body, jnp.zeros((L,), jnp.float32))
            acc_v[:] = tot
            n_all = plsc.load_gather(acc_v, [jnp.zeros((L,), jnp.int32)])
            m_all = plsc.load_gather(acc_v, [jnp.ones((L,), jnp.int32)])
            out_v[:] = n_all / (2.0 * m_all + 0.0001)
            pltpu.sync_copy(out_v, out_hbm)


@jax.jit
def kernel(flow, mask, index, ids, index2, ids2):
    tab = jnp.pad(jnp.stack([ids, ids2, index, index2, mask]),
                  ((0, 0), (0, 0), (0, KP - K)))
    flow_flat = flow.reshape(-1)

    per_batch = [
        pltpu.VMEM((KP,), jnp.int32),
        pltpu.VMEM((KP,), jnp.int32),
        pltpu.VMEM((KP,), jnp.int32),
        pltpu.VMEM((KP,), jnp.int32),
        pltpu.VMEM((KP,), jnp.int32),
        pltpu.VMEM((VT,), jnp.int32),
        pltpu.VMEM((VT,), jnp.int32),
        pltpu.VMEM((KP,), jnp.int32),
        pltpu.VMEM((KP,), jnp.int32),
        pltpu.VMEM((KP,), jnp.float32),
        pltpu.VMEM((KP,), jnp.float32),
        pltpu.VMEM((L,), jnp.int32),
    ]
    mesh = plsc.VectorSubcoreMesh(core_axis_name="c", subcore_axis_name="s")
    run = functools.partial(
        pl.kernel, mesh=mesh,
        compiler_params=pltpu.CompilerParams(needs_layout_passes=False),
        out_type=jax.ShapeDtypeStruct((L,), jnp.float32),
        scratch_types=per_batch * NB + [
            pltpu.VMEM((L,), jnp.float32),
            pltpu.VMEM((NS * L,), jnp.float32),
            pltpu.VMEM((L,), jnp.float32),
            pltpu.VMEM_SHARED((NS * L,), jnp.float32),
            pltpu.SemaphoreType.DMA,
            pltpu.SemaphoreType.DMA,
        ],
    )(_sc_body)
    out = run(flow_flat, tab)
    return out[0]
